# Initial kernel scaffold; baseline (speedup 1.0000x reference)
import jax
import jax.numpy as jnp
from jax.experimental import pallas as pl


def kernel(x, w1, b1, g1, be1, w2, b2, g2, be2):
    raise NotImplementedError("write your pallas kernel here")


# speedup vs baseline: 1.2870x; 1.2870x over previous
import functools

import jax
import jax.numpy as jnp
from jax import lax
from jax.experimental import pallas as pl
from jax.experimental.pallas import tpu as pltpu

_EPS = 1e-5
_VMEM_LIMIT = 64 * 1024 * 1024 * 3 // 4


def _fetch_rows(src_hbm, dst_ref, sem, base, n, h, th, n_h, c0, W):
    row0 = h * th
    main = pltpu.make_async_copy(
        src_hbm.at[n, pl.ds(row0, th)],
        dst_ref.at[pl.ds(1, th), pl.ds(c0, W)],
        sem.at[base])
    main.start()

    @pl.when(h > 0)
    def _():
        cp = pltpu.make_async_copy(
            src_hbm.at[n, pl.ds(row0 - 1, 1)],
            dst_ref.at[pl.ds(0, 1), pl.ds(c0, W)],
            sem.at[base + 1])
        cp.start()
        cp.wait()

    @pl.when(h < n_h - 1)
    def _():
        cp = pltpu.make_async_copy(
            src_hbm.at[n, pl.ds(row0 + th, 1)],
            dst_ref.at[pl.ds(th + 1, 1), pl.ds(c0, W)],
            sem.at[base + 2])
        cp.start()
        cp.wait()

    main.wait()


def _im2col(src_ref, col_ref, K, th, W, C, cast):
    for di in range(K):
        for dj in range(K):
            t = di * K + dj
            patch = src_ref[di:di + th, dj:dj + W]
            if cast:
                patch = patch.astype(jnp.bfloat16)
            col_ref[:, t * C:(t + 1) * C] = patch.reshape(th * W, C)


def _tile_stats(acc, stat_ref, m_tile):
    s = jnp.sum(acc, axis=0, keepdims=True)
    mu = s * (1.0 / m_tile)
    d = acc - mu
    stat_ref[0, 0, 0:1, :] = s
    stat_ref[0, 0, 1:2, :] = jnp.sum(d * d, axis=0, keepdims=True)


def _conv1_kernel(x_hbm, w_ref, y_ref, stat_ref, xh_ref, col_ref, sem,
                  *, K, th, n_h, W, C):
    n = pl.program_id(0)
    h = pl.program_id(1)
    pad = (K - 1) // 2
    Wp = W + 2 * pad
    _fetch_rows(x_hbm, xh_ref, sem, 0, n, h, th, n_h, pad, W)
    xh_ref[:, 0:pad] = jnp.zeros((th + 2 * pad, pad, C), jnp.float32)
    xh_ref[:, Wp - pad:Wp] = jnp.zeros((th + 2 * pad, pad, C), jnp.float32)

    @pl.when(h == 0)
    def _():
        xh_ref[0:pad] = jnp.zeros((pad, Wp, C), jnp.float32)

    @pl.when(h == n_h - 1)
    def _():
        xh_ref[th + pad:th + 2 * pad] = jnp.zeros((pad, Wp, C), jnp.float32)

    _im2col(xh_ref, col_ref, K, th, W, C, cast=True)
    acc = jnp.dot(col_ref[...], w_ref[...], preferred_element_type=jnp.float32)
    y_ref[0] = acc.reshape(th, W, C)
    _tile_stats(acc, stat_ref, th * W)


def _mid_kernel(y1_hbm, x_hbm, w_ref, s1_ref, t1_ref,
                y2_ref, x1_ref, stat_ref,
                y1h_ref, xh_ref, x1p_ref, col_ref, sem,
                *, K, th, n_h, W, C):
    n = pl.program_id(0)
    h = pl.program_id(1)
    pad = (K - 1) // 2
    Wp = W + 2 * pad
    _fetch_rows(y1_hbm, y1h_ref, sem, 0, n, h, th, n_h, 0, W)
    _fetch_rows(x_hbm, xh_ref, sem, 3, n, h, th, n_h, 0, W)

    x1 = jnp.maximum(y1h_ref[...] * s1_ref[...] + t1_ref[...], 0.0) + xh_ref[...]
    x1_ref[0] = x1[pad:pad + th]
    x1p_ref[:, pad:pad + W] = x1.astype(jnp.bfloat16)
    x1p_ref[:, 0:pad] = jnp.zeros((th + 2 * pad, pad, C), jnp.bfloat16)
    x1p_ref[:, Wp - pad:Wp] = jnp.zeros((th + 2 * pad, pad, C), jnp.bfloat16)

    @pl.when(h == 0)
    def _():
        x1p_ref[0:pad] = jnp.zeros((pad, Wp, C), jnp.bfloat16)

    @pl.when(h == n_h - 1)
    def _():
        x1p_ref[th + pad:th + 2 * pad] = jnp.zeros((pad, Wp, C), jnp.bfloat16)

    _im2col(x1p_ref, col_ref, K, th, W, C, cast=False)
    acc = jnp.dot(col_ref[...], w_ref[...], preferred_element_type=jnp.float32)
    y2_ref[0] = acc.reshape(th, W, C)
    _tile_stats(acc, stat_ref, th * W)


def _final_kernel(y2_ref, x1_ref, s2_ref, t2_ref, out_ref):
    out_ref[...] = (jnp.maximum(y2_ref[...] * s2_ref[...] + t2_ref[...], 0.0)
                    + x1_ref[...])


def _bn_affine(stats, gamma, beta, m_tile, count):
    sums = stats[:, :, 0, :]
    m2s = stats[:, :, 1, :]
    mean = jnp.sum(sums, axis=(0, 1)) / count
    tile_mean = sums / m_tile
    m2 = jnp.sum(m2s, axis=(0, 1)) + m_tile * jnp.sum(
        jnp.square(tile_mean - mean), axis=(0, 1))
    var = jnp.maximum(m2 / count, 0.0)
    scale = gamma * lax.rsqrt(var + _EPS)
    shift = beta - mean * scale
    C = gamma.shape[0]
    return (scale.reshape(1, C).astype(jnp.float32),
            shift.reshape(1, C).astype(jnp.float32))


def kernel(x, w1, b1, g1, be1, w2, b2, g2, be2):
    N, C, H, W = x.shape
    K = w1.shape[-1]
    pad = (K - 1) // 2
    th = min(32, H)
    n_h = H // th
    Wp = W + 2 * pad
    kkc = K * K * C
    m_tile = th * W
    count = N * H * W

    x_nhwc = jnp.transpose(x, (0, 2, 3, 1)).astype(jnp.float32)
    w1f = jnp.transpose(w1, (2, 3, 1, 0)).reshape(kkc, C).astype(jnp.bfloat16)
    w2f = jnp.transpose(w2, (2, 3, 1, 0)).reshape(kkc, C).astype(jnp.bfloat16)

    cparams = pltpu.CompilerParams(
        dimension_semantics=("parallel", "parallel"),
        vmem_limit_bytes=_VMEM_LIMIT)
    conv_flops = 2 * N * H * W * kkc * C

    y1, st1 = pl.pallas_call(
        functools.partial(_conv1_kernel, K=K, th=th, n_h=n_h, W=W, C=C),
        out_shape=[jax.ShapeDtypeStruct((N, H, W, C), jnp.float32),
                   jax.ShapeDtypeStruct((N, n_h, 2, C), jnp.float32)],
        grid=(N, n_h),
        in_specs=[pl.BlockSpec(memory_space=pl.ANY),
                  pl.BlockSpec((kkc, C), lambda n, h: (0, 0))],
        out_specs=[pl.BlockSpec((1, th, W, C), lambda n, h: (n, h, 0, 0)),
                   pl.BlockSpec((1, 1, 2, C), lambda n, h: (n, h, 0, 0))],
        scratch_shapes=[pltpu.VMEM((th + 2 * pad, Wp, C), jnp.float32),
                        pltpu.VMEM((th * W, kkc), jnp.bfloat16),
                        pltpu.SemaphoreType.DMA((3,))],
        compiler_params=cparams,
        cost_estimate=pl.CostEstimate(
            flops=conv_flops, transcendentals=0,
            bytes_accessed=2 * N * H * W * C * 4),
    )(x_nhwc, w1f)

    s1, t1 = _bn_affine(st1, g1, be1, m_tile, count)

    y2, x1, st2 = pl.pallas_call(
        functools.partial(_mid_kernel, K=K, th=th, n_h=n_h, W=W, C=C),
        out_shape=[jax.ShapeDtypeStruct((N, H, W, C), jnp.float32),
                   jax.ShapeDtypeStruct((N, H, W, C), jnp.float32),
                   jax.ShapeDtypeStruct((N, n_h, 2, C), jnp.float32)],
        grid=(N, n_h),
        in_specs=[pl.BlockSpec(memory_space=pl.ANY),
                  pl.BlockSpec(memory_space=pl.ANY),
                  pl.BlockSpec((kkc, C), lambda n, h: (0, 0)),
                  pl.BlockSpec((1, C), lambda n, h: (0, 0)),
                  pl.BlockSpec((1, C), lambda n, h: (0, 0))],
        out_specs=[pl.BlockSpec((1, th, W, C), lambda n, h: (n, h, 0, 0)),
                   pl.BlockSpec((1, th, W, C), lambda n, h: (n, h, 0, 0)),
                   pl.BlockSpec((1, 1, 2, C), lambda n, h: (n, h, 0, 0))],
        scratch_shapes=[pltpu.VMEM((th + 2 * pad, W, C), jnp.float32),
                        pltpu.VMEM((th + 2 * pad, W, C), jnp.float32),
                        pltpu.VMEM((th + 2 * pad, Wp, C), jnp.bfloat16),
                        pltpu.VMEM((th * W, kkc), jnp.bfloat16),
                        pltpu.SemaphoreType.DMA((6,))],
        compiler_params=cparams,
        cost_estimate=pl.CostEstimate(
            flops=conv_flops, transcendentals=0,
            bytes_accessed=4 * N * H * W * C * 4),
    )(y1, x_nhwc, w2f, s1, t1)

    s2, t2 = _bn_affine(st2, g2, be2, m_tile, count)

    out = pl.pallas_call(
        functools.partial(_final_kernel),
        out_shape=jax.ShapeDtypeStruct((N, H, W, C), jnp.float32),
        grid=(N, n_h),
        in_specs=[pl.BlockSpec((1, th, W, C), lambda n, h: (n, h, 0, 0)),
                  pl.BlockSpec((1, th, W, C), lambda n, h: (n, h, 0, 0)),
                  pl.BlockSpec((1, C), lambda n, h: (0, 0)),
                  pl.BlockSpec((1, C), lambda n, h: (0, 0))],
        out_specs=pl.BlockSpec((1, th, W, C), lambda n, h: (n, h, 0, 0)),
        compiler_params=cparams,
        cost_estimate=pl.CostEstimate(
            flops=3 * N * H * W * C, transcendentals=0,
            bytes_accessed=3 * N * H * W * C * 4),
    )(y2, x1, s2, t2)

    return jnp.transpose(out, (0, 3, 1, 2))


# speedup vs baseline: 1.5363x; 1.1937x over previous
import functools

import jax
import jax.numpy as jnp
from jax import lax
from jax.experimental import pallas as pl
from jax.experimental.pallas import tpu as pltpu

_EPS = 1e-5
_VMEM_LIMIT = 64 * 1024 * 1024 * 3 // 4


def _fetch_rows(src_hbm, dst_ref, sem, base, n, h, th, n_h, c0, W):
    row0 = h * th
    main = pltpu.make_async_copy(
        src_hbm.at[n, pl.ds(row0, th)],
        dst_ref.at[pl.ds(1, th), pl.ds(c0, W)],
        sem.at[base])
    main.start()

    @pl.when(h > 0)
    def _():
        cp = pltpu.make_async_copy(
            src_hbm.at[n, pl.ds(row0 - 1, 1)],
            dst_ref.at[pl.ds(0, 1), pl.ds(c0, W)],
            sem.at[base + 1])
        cp.start()
        cp.wait()

    @pl.when(h < n_h - 1)
    def _():
        cp = pltpu.make_async_copy(
            src_hbm.at[n, pl.ds(row0 + th, 1)],
            dst_ref.at[pl.ds(th + 1, 1), pl.ds(c0, W)],
            sem.at[base + 2])
        cp.start()
        cp.wait()

    main.wait()


def _zero_pads(ref, h, n_h, K, th, W, Wq, dtype):
    pad = (K - 1) // 2
    ref[:, 0:pad] = jnp.zeros((th + 2 * pad, pad, ref.shape[2]), dtype)
    ref[:, pad + W:Wq] = jnp.zeros((th + 2 * pad, Wq - pad - W, ref.shape[2]),
                                   dtype)

    @pl.when(h == 0)
    def _():
        ref[0:pad] = jnp.zeros((pad, Wq, ref.shape[2]), dtype)

    @pl.when(h == n_h - 1)
    def _():
        ref[th + pad:th + 2 * pad] = jnp.zeros((pad, Wq, ref.shape[2]), dtype)


def _conv_rowtaps(src_ref, col_ref, w_ref, p_ref, K, th, W, Wq, C, cast):
    for di in range(K):
        patch = src_ref[di:di + th]
        if cast:
            patch = patch.astype(jnp.bfloat16)
        col_ref[:, di * C:(di + 1) * C] = patch.reshape(th * Wq, C)
    p_ref[...] = jnp.dot(
        col_ref[...], w_ref[...],
        preferred_element_type=jnp.float32).reshape(th, Wq, K * C)
    y = p_ref[:, 0:W, 0:C]
    for dj in range(1, K):
        y = y + p_ref[:, dj:dj + W, dj * C:(dj + 1) * C]
    return y


def _tile_stats(y, stat_ref, m_tile, C):
    acc = y.reshape(m_tile, C)
    s = jnp.sum(acc, axis=0, keepdims=True)
    mu = s * (1.0 / m_tile)
    d = acc - mu
    stat_ref[0, 0, 0:1, :] = s
    stat_ref[0, 0, 1:2, :] = jnp.sum(d * d, axis=0, keepdims=True)


def _conv1_kernel(x_hbm, w_ref, y_ref, stat_ref, xh_ref, col_ref, p_ref, sem,
                  *, K, th, n_h, W, Wq, C):
    n = pl.program_id(0)
    h = pl.program_id(1)
    pad = (K - 1) // 2
    _fetch_rows(x_hbm, xh_ref, sem, 0, n, h, th, n_h, pad, W)
    _zero_pads(xh_ref, h, n_h, K, th, W, Wq, jnp.float32)
    y = _conv_rowtaps(xh_ref, col_ref, w_ref, p_ref, K, th, W, Wq, C,
                      cast=True)
    y_ref[0] = y.astype(jnp.bfloat16)
    _tile_stats(y, stat_ref, th * W, C)


def _mid_kernel(y1_hbm, x_hbm, w_ref, s1_ref, t1_ref,
                y2_ref, x1_ref, stat_ref,
                y1h_ref, xh_ref, x1p_ref, col_ref, p_ref, sem,
                *, K, th, n_h, W, Wq, C):
    n = pl.program_id(0)
    h = pl.program_id(1)
    pad = (K - 1) // 2
    _fetch_rows(y1_hbm, y1h_ref, sem, 0, n, h, th, n_h, 0, W)
    _fetch_rows(x_hbm, xh_ref, sem, 3, n, h, th, n_h, 0, W)

    x1 = (jnp.maximum(y1h_ref[...].astype(jnp.float32) * s1_ref[...]
                      + t1_ref[...], 0.0) + xh_ref[...])
    x1_ref[0] = x1[pad:pad + th].astype(jnp.bfloat16)
    x1p_ref[:, pad:pad + W] = x1.astype(jnp.bfloat16)
    _zero_pads(x1p_ref, h, n_h, K, th, W, Wq, jnp.bfloat16)
    y = _conv_rowtaps(x1p_ref, col_ref, w_ref, p_ref, K, th, W, Wq, C,
                      cast=False)
    y2_ref[0] = y.astype(jnp.bfloat16)
    _tile_stats(y, stat_ref, th * W, C)


def _final_kernel(y2_ref, x1_ref, s2_ref, t2_ref, out_ref):
    out_ref[...] = (jnp.maximum(y2_ref[...].astype(jnp.float32) * s2_ref[...]
                                + t2_ref[...], 0.0)
                    + x1_ref[...].astype(jnp.float32))


def _bn_affine(stats, gamma, beta, m_tile, count):
    sums = stats[:, :, 0, :]
    m2s = stats[:, :, 1, :]
    mean = jnp.sum(sums, axis=(0, 1)) / count
    tile_mean = sums / m_tile
    m2 = jnp.sum(m2s, axis=(0, 1)) + m_tile * jnp.sum(
        jnp.square(tile_mean - mean), axis=(0, 1))
    var = jnp.maximum(m2 / count, 0.0)
    scale = gamma * lax.rsqrt(var + _EPS)
    shift = beta - mean * scale
    C = gamma.shape[0]
    return (scale.reshape(1, C).astype(jnp.float32),
            shift.reshape(1, C).astype(jnp.float32))


def kernel(x, w1, b1, g1, be1, w2, b2, g2, be2):
    N, C, H, W = x.shape
    K = w1.shape[-1]
    pad = (K - 1) // 2
    th = min(32, H)
    n_h = H // th
    Wq = ((W + 2 * pad + 7) // 8) * 8
    m_tile = th * W
    count = N * H * W

    x_nhwc = jnp.transpose(x, (0, 2, 3, 1)).astype(jnp.float32)
    w1f = jnp.transpose(w1, (2, 1, 3, 0)).reshape(K * C, K * C).astype(
        jnp.bfloat16)
    w2f = jnp.transpose(w2, (2, 1, 3, 0)).reshape(K * C, K * C).astype(
        jnp.bfloat16)

    cparams = pltpu.CompilerParams(
        dimension_semantics=("parallel", "parallel"),
        vmem_limit_bytes=_VMEM_LIMIT)
    conv_flops = 2 * N * H * Wq * K * C * K * C

    y1, st1 = pl.pallas_call(
        functools.partial(_conv1_kernel, K=K, th=th, n_h=n_h, W=W, Wq=Wq, C=C),
        out_shape=[jax.ShapeDtypeStruct((N, H, W, C), jnp.bfloat16),
                   jax.ShapeDtypeStruct((N, n_h, 2, C), jnp.float32)],
        grid=(N, n_h),
        in_specs=[pl.BlockSpec(memory_space=pl.ANY),
                  pl.BlockSpec((K * C, K * C), lambda n, h: (0, 0))],
        out_specs=[pl.BlockSpec((1, th, W, C), lambda n, h: (n, h, 0, 0)),
                   pl.BlockSpec((1, 1, 2, C), lambda n, h: (n, h, 0, 0))],
        scratch_shapes=[pltpu.VMEM((th + 2 * pad, Wq, C), jnp.float32),
                        pltpu.VMEM((th * Wq, K * C), jnp.bfloat16),
                        pltpu.VMEM((th, Wq, K * C), jnp.float32),
                        pltpu.SemaphoreType.DMA((3,))],
        compiler_params=cparams,
        cost_estimate=pl.CostEstimate(
            flops=conv_flops, transcendentals=0,
            bytes_accessed=N * H * W * C * 6),
    )(x_nhwc, w1f)

    s1, t1 = _bn_affine(st1, g1, be1, m_tile, count)

    y2, x1, st2 = pl.pallas_call(
        functools.partial(_mid_kernel, K=K, th=th, n_h=n_h, W=W, Wq=Wq, C=C),
        out_shape=[jax.ShapeDtypeStruct((N, H, W, C), jnp.bfloat16),
                   jax.ShapeDtypeStruct((N, H, W, C), jnp.bfloat16),
                   jax.ShapeDtypeStruct((N, n_h, 2, C), jnp.float32)],
        grid=(N, n_h),
        in_specs=[pl.BlockSpec(memory_space=pl.ANY),
                  pl.BlockSpec(memory_space=pl.ANY),
                  pl.BlockSpec((K * C, K * C), lambda n, h: (0, 0)),
                  pl.BlockSpec((1, C), lambda n, h: (0, 0)),
                  pl.BlockSpec((1, C), lambda n, h: (0, 0))],
        out_specs=[pl.BlockSpec((1, th, W, C), lambda n, h: (n, h, 0, 0)),
                   pl.BlockSpec((1, th, W, C), lambda n, h: (n, h, 0, 0)),
                   pl.BlockSpec((1, 1, 2, C), lambda n, h: (n, h, 0, 0))],
        scratch_shapes=[pltpu.VMEM((th + 2 * pad, W, C), jnp.bfloat16),
                        pltpu.VMEM((th + 2 * pad, W, C), jnp.float32),
                        pltpu.VMEM((th + 2 * pad, Wq, C), jnp.bfloat16),
                        pltpu.VMEM((th * Wq, K * C), jnp.bfloat16),
                        pltpu.VMEM((th, Wq, K * C), jnp.float32),
                        pltpu.SemaphoreType.DMA((6,))],
        compiler_params=cparams,
        cost_estimate=pl.CostEstimate(
            flops=conv_flops, transcendentals=0,
            bytes_accessed=N * H * W * C * 10),
    )(y1, x_nhwc, w2f, s1, t1)

    s2, t2 = _bn_affine(st2, g2, be2, m_tile, count)

    out = pl.pallas_call(
        _final_kernel,
        out_shape=jax.ShapeDtypeStruct((N, H, W, C), jnp.float32),
        grid=(N, n_h),
        in_specs=[pl.BlockSpec((1, th, W, C), lambda n, h: (n, h, 0, 0)),
                  pl.BlockSpec((1, th, W, C), lambda n, h: (n, h, 0, 0)),
                  pl.BlockSpec((1, C), lambda n, h: (0, 0)),
                  pl.BlockSpec((1, C), lambda n, h: (0, 0))],
        out_specs=pl.BlockSpec((1, th, W, C), lambda n, h: (n, h, 0, 0)),
        compiler_params=cparams,
        cost_estimate=pl.CostEstimate(
            flops=3 * N * H * W * C, transcendentals=0,
            bytes_accessed=2 * N * H * W * C * 4),
    )(y2, x1, s2, t2)

    return jnp.transpose(out, (0, 3, 1, 2))


# speedup vs baseline: 3.0147x; 1.9623x over previous
import functools

import jax
import jax.numpy as jnp
from jax import lax
from jax.experimental import pallas as pl
from jax.experimental.pallas import tpu as pltpu

_EPS = 1e-5
_VMEM_LIMIT = 64 * 1024 * 1024 * 3 // 4


def _build_taps(src, col_ref, K, H, W, Wq, C):
    pad = (K - 1) // 2
    col_ref[:, 0:pad, :] = jnp.zeros((H, pad, K * C), jnp.bfloat16)
    col_ref[:, pad + W:Wq, :] = jnp.zeros((H, Wq - pad - W, K * C),
                                          jnp.bfloat16)
    for di in range(K):
        r0 = max(0, pad - di)
        r1 = min(H, H + pad - di)
        if r0 > 0:
            col_ref[0:r0, pad:pad + W, di * C:(di + 1) * C] = jnp.zeros(
                (r0, W, C), jnp.bfloat16)
        if r1 < H:
            col_ref[r1:H, pad:pad + W, di * C:(di + 1) * C] = jnp.zeros(
                (H - r1, W, C), jnp.bfloat16)
        col_ref[r0:r1, pad:pad + W, di * C:(di + 1) * C] = (
            src[r0 + di - pad:r1 + di - pad])


def _conv_epilogue(col_ref, w_ref, p_ref, K, H, W, Wq, C):
    p_ref[...] = jnp.dot(
        col_ref[...].reshape(H * Wq, K * C), w_ref[...],
        preferred_element_type=jnp.float32).reshape(H, Wq, K * C)
    y = p_ref[:, 0:W, 0:C]
    for dj in range(1, K):
        y = y + p_ref[:, dj:dj + W, dj * C:(dj + 1) * C]
    return y


def _tile_stats(y, stat_ref, m_tile, C):
    acc = y.reshape(m_tile, C)
    s = jnp.sum(acc, axis=0, keepdims=True)
    mu = s * (1.0 / m_tile)
    d = acc - mu
    stat_ref[0, 0:1, :] = s
    stat_ref[0, 1:2, :] = jnp.sum(d * d, axis=0, keepdims=True)


def _conv1_kernel(x_ref, w_ref, y_ref, stat_ref, col_ref, p_ref,
                  *, K, H, W, Wq, C):
    _build_taps(x_ref[0].astype(jnp.bfloat16), col_ref, K, H, W, Wq, C)
    y = _conv_epilogue(col_ref, w_ref, p_ref, K, H, W, Wq, C)
    y_ref[0] = y.astype(jnp.bfloat16)
    _tile_stats(y, stat_ref, H * W, C)


def _mid_kernel(y1_ref, x_ref, w_ref, s1_ref, t1_ref,
                y2_ref, x1_ref, stat_ref, col_ref, p_ref,
                *, K, H, W, Wq, C):
    x1 = (jnp.maximum(y1_ref[0].astype(jnp.float32) * s1_ref[...]
                      + t1_ref[...], 0.0) + x_ref[0])
    x1_ref[0] = x1.astype(jnp.bfloat16)
    _build_taps(x1_ref[0], col_ref, K, H, W, Wq, C)
    y = _conv_epilogue(col_ref, w_ref, p_ref, K, H, W, Wq, C)
    y2_ref[0] = y.astype(jnp.bfloat16)
    _tile_stats(y, stat_ref, H * W, C)


def _final_kernel(y2_ref, x1_ref, s2_ref, t2_ref, out_ref):
    out_ref[...] = (jnp.maximum(y2_ref[...].astype(jnp.float32) * s2_ref[...]
                                + t2_ref[...], 0.0)
                    + x1_ref[...].astype(jnp.float32))


def _bn_affine(stats, gamma, beta, m_tile, count):
    sums = stats[:, 0, :]
    m2s = stats[:, 1, :]
    mean = jnp.sum(sums, axis=0) / count
    tile_mean = sums / m_tile
    m2 = jnp.sum(m2s, axis=0) + m_tile * jnp.sum(
        jnp.square(tile_mean - mean), axis=0)
    var = jnp.maximum(m2 / count, 0.0)
    scale = gamma * lax.rsqrt(var + _EPS)
    shift = beta - mean * scale
    C = gamma.shape[0]
    return (scale.reshape(1, C).astype(jnp.float32),
            shift.reshape(1, C).astype(jnp.float32))


def kernel(x, w1, b1, g1, be1, w2, b2, g2, be2):
    N, C, H, W = x.shape
    K = w1.shape[-1]
    pad = (K - 1) // 2
    Wq = ((W + 2 * pad + 7) // 8) * 8
    m_tile = H * W
    count = N * H * W

    x_nhwc = jnp.transpose(x, (0, 2, 3, 1)).astype(jnp.float32)
    w1f = jnp.transpose(w1, (2, 1, 3, 0)).reshape(K * C, K * C).astype(
        jnp.bfloat16)
    w2f = jnp.transpose(w2, (2, 1, 3, 0)).reshape(K * C, K * C).astype(
        jnp.bfloat16)

    cparams = pltpu.CompilerParams(
        dimension_semantics=("parallel",),
        vmem_limit_bytes=_VMEM_LIMIT)
    conv_flops = 2 * N * H * Wq * K * C * K * C

    y1, st1 = pl.pallas_call(
        functools.partial(_conv1_kernel, K=K, H=H, W=W, Wq=Wq, C=C),
        out_shape=[jax.ShapeDtypeStruct((N, H, W, C), jnp.bfloat16),
                   jax.ShapeDtypeStruct((N, 2, C), jnp.float32)],
        grid=(N,),
        in_specs=[pl.BlockSpec((1, H, W, C), lambda n: (n, 0, 0, 0)),
                  pl.BlockSpec((K * C, K * C), lambda n: (0, 0))],
        out_specs=[pl.BlockSpec((1, H, W, C), lambda n: (n, 0, 0, 0)),
                   pl.BlockSpec((1, 2, C), lambda n: (n, 0, 0))],
        scratch_shapes=[pltpu.VMEM((H, Wq, K * C), jnp.bfloat16),
                        pltpu.VMEM((H, Wq, K * C), jnp.float32)],
        compiler_params=cparams,
        cost_estimate=pl.CostEstimate(
            flops=conv_flops, transcendentals=0,
            bytes_accessed=N * H * W * C * 6),
    )(x_nhwc, w1f)

    s1, t1 = _bn_affine(st1, g1, be1, m_tile, count)

    y2, x1, st2 = pl.pallas_call(
        functools.partial(_mid_kernel, K=K, H=H, W=W, Wq=Wq, C=C),
        out_shape=[jax.ShapeDtypeStruct((N, H, W, C), jnp.bfloat16),
                   jax.ShapeDtypeStruct((N, H, W, C), jnp.bfloat16),
                   jax.ShapeDtypeStruct((N, 2, C), jnp.float32)],
        grid=(N,),
        in_specs=[pl.BlockSpec((1, H, W, C), lambda n: (n, 0, 0, 0)),
                  pl.BlockSpec((1, H, W, C), lambda n: (n, 0, 0, 0)),
                  pl.BlockSpec((K * C, K * C), lambda n: (0, 0)),
                  pl.BlockSpec((1, C), lambda n: (0, 0)),
                  pl.BlockSpec((1, C), lambda n: (0, 0))],
        out_specs=[pl.BlockSpec((1, H, W, C), lambda n: (n, 0, 0, 0)),
                   pl.BlockSpec((1, H, W, C), lambda n: (n, 0, 0, 0)),
                   pl.BlockSpec((1, 2, C), lambda n: (n, 0, 0))],
        scratch_shapes=[pltpu.VMEM((H, Wq, K * C), jnp.bfloat16),
                        pltpu.VMEM((H, Wq, K * C), jnp.float32)],
        compiler_params=cparams,
        cost_estimate=pl.CostEstimate(
            flops=conv_flops, transcendentals=0,
            bytes_accessed=N * H * W * C * 10),
    )(y1, x_nhwc, w2f, s1, t1)

    s2, t2 = _bn_affine(st2, g2, be2, m_tile, count)

    out = pl.pallas_call(
        _final_kernel,
        out_shape=jax.ShapeDtypeStruct((N, H, W, C), jnp.float32),
        grid=(N,),
        in_specs=[pl.BlockSpec((1, H, W, C), lambda n: (n, 0, 0, 0)),
                  pl.BlockSpec((1, H, W, C), lambda n: (n, 0, 0, 0)),
                  pl.BlockSpec((1, C), lambda n: (0, 0)),
                  pl.BlockSpec((1, C), lambda n: (0, 0))],
        out_specs=pl.BlockSpec((1, H, W, C), lambda n: (n, 0, 0, 0)),
        compiler_params=cparams,
        cost_estimate=pl.CostEstimate(
            flops=3 * N * H * W * C, transcendentals=0,
            bytes_accessed=2 * N * H * W * C * 4),
    )(y2, x1, s2, t2)

    return jnp.transpose(out, (0, 3, 1, 2))
